# baseline (speedup 1.0000x reference)
import functools

import jax
import jax.numpy as jnp
from jax import lax
from jax.experimental import pallas as pl
from jax.experimental.pallas import tpu as pltpu
from jax.experimental.pallas import tpu_sc as plsc

_SEQ = 8192
_D = 1024
_NC, _NS = 2, 16
_NW = _NC * _NS
_BPW = _SEQ // _NW
_CH = 16
_NCHUNK = _BPW // _CH
_NBUF = 6
_GDEPTH = 3

_mesh = plsc.VectorSubcoreMesh(core_axis_name="c", subcore_axis_name="s")


@functools.partial(
    pl.kernel,
    out_type=jax.ShapeDtypeStruct((1, _SEQ, _D), jnp.float32),
    mesh=_mesh,
    scratch_types=[
        pltpu.VMEM((_BPW,), jnp.int32),
        [pltpu.VMEM((_CH, _D), jnp.float32) for _ in range(_NBUF)],
        [pltpu.SemaphoreType.DMA for _ in range(_NBUF)],
        [pltpu.SemaphoreType.DMA for _ in range(_NBUF)],
    ],
)
def _pe_gather(table_hbm, idx_hbm, out_hbm, idx_v, bufs, gsems, ssems):
    wid = lax.axis_index("s") * _NC + lax.axis_index("c")
    base = wid * _BPW
    pltpu.sync_copy(idx_hbm.at[pl.ds(base, _BPW)], idx_v)

    gathers = [None] * _NBUF
    stores = [None] * _NCHUNK
    for c in range(min(_GDEPTH, _NCHUNK)):
        gathers[c % _NBUF] = pltpu.async_copy(
            table_hbm.at[idx_v.at[pl.ds(c * _CH, _CH)]], bufs[c % _NBUF],
            gsems[c % _NBUF])
    for c in range(_NCHUNK):
        b = c % _NBUF
        gathers[b].wait()
        stores[c] = pltpu.async_copy(
            bufs[b], out_hbm.at[0, pl.ds(base + c * _CH, _CH)], ssems[b])
        nc = c + _GDEPTH
        if nc < _NCHUNK:
            nb = nc % _NBUF
            prev = nc - _NBUF
            if prev >= 0:
                stores[prev].wait()
            gathers[nb] = pltpu.async_copy(
                table_hbm.at[idx_v.at[pl.ds(nc * _CH, _CH)]], bufs[nb],
                gsems[nb])
    for c in range(max(0, _NCHUNK - _NBUF), _NCHUNK):
        if stores[c] is not None:
            stores[c].wait()


def kernel(x, pe_table, position_ids):
    del x
    idx = position_ids.reshape(_SEQ).astype(jnp.int32)
    return _pe_gather(pe_table, idx)


# speedup vs baseline: 1.0172x; 1.0172x over previous
import functools

import jax
import jax.numpy as jnp
from jax import lax
from jax.experimental import pallas as pl
from jax.experimental.pallas import tpu as pltpu
from jax.experimental.pallas import tpu_sc as plsc

_SEQ = 8192
_D = 1024
_NC, _NS = 2, 16
_NW = _NC * _NS
_BPW = _SEQ // _NW
_CH = 16
_NCHUNK = _BPW // _CH
_NBUF = 7
_GDEPTH = 4

_mesh = plsc.VectorSubcoreMesh(core_axis_name="c", subcore_axis_name="s")


@functools.partial(
    pl.kernel,
    out_type=jax.ShapeDtypeStruct((1, _SEQ, _D), jnp.float32),
    mesh=_mesh,
    scratch_types=[
        pltpu.VMEM((_BPW,), jnp.int32),
        [pltpu.VMEM((_CH, _D), jnp.float32) for _ in range(_NBUF)],
        [pltpu.SemaphoreType.DMA for _ in range(_NBUF)],
        [pltpu.SemaphoreType.DMA for _ in range(_NBUF)],
    ],
)
def _pe_gather(table_hbm, idx_hbm, out_hbm, idx_v, bufs, gsems, ssems):
    wid = lax.axis_index("s") * _NC + lax.axis_index("c")
    base = wid * _BPW
    pltpu.sync_copy(idx_hbm.at[pl.ds(base, _BPW)], idx_v)

    gathers = [None] * _NBUF
    stores = [None] * _NCHUNK
    for c in range(min(_GDEPTH, _NCHUNK)):
        gathers[c % _NBUF] = pltpu.async_copy(
            table_hbm.at[idx_v.at[pl.ds(c * _CH, _CH)]], bufs[c % _NBUF],
            gsems[c % _NBUF])
    for c in range(_NCHUNK):
        b = c % _NBUF
        gathers[b].wait()
        stores[c] = pltpu.async_copy(
            bufs[b], out_hbm.at[0, pl.ds(base + c * _CH, _CH)], ssems[b])
        nc = c + _GDEPTH
        if nc < _NCHUNK:
            nb = nc % _NBUF
            prev = nc - _NBUF
            if prev >= 0:
                stores[prev].wait()
            gathers[nb] = pltpu.async_copy(
                table_hbm.at[idx_v.at[pl.ds(nc * _CH, _CH)]], bufs[nb],
                gsems[nb])
    for c in range(max(0, _NCHUNK - _NBUF), _NCHUNK):
        if stores[c] is not None:
            stores[c].wait()


def kernel(x, pe_table, position_ids):
    del x
    idx = position_ids.reshape(_SEQ).astype(jnp.int32)
    return _pe_gather(pe_table, idx)
